# baseline (speedup 1.0000x reference)
import functools

import jax
import jax.numpy as jnp
from jax import lax
from jax.experimental import pallas as pl
from jax.experimental.pallas import tpu as pltpu
from jax.experimental.pallas import tpu_sc as plsc

_BATCH = 1024
_SEQ = 200
_N = _BATCH * _SEQ
_NC = 2
_NS = 16
_NW = _NC * _NS
_PER_W = _N // _NW
_CHUNK = 128
_NCHUNK = _PER_W // _CHUNK
_HVEC = _PER_W // 16
_EDIM = 64
_PDIM = 128
_MDIM = 512
_MULT_A = 36313
_MULT_B = 27191
_MOD = 999999


def _sc_body(nchunk, tok_hbm, table_hbm, out_hbm, tok_v, idx_v, rows_v, sem):
    wid = lax.axis_index("s") * _NC + lax.axis_index("c")
    base = wid * nchunk * _CHUNK

    def tok_row(r, _):
        pltpu.sync_copy(tok_hbm.at[wid * nchunk + r],
                        tok_v.at[pl.ds(8 + r * _PDIM, _PDIM)])
        return 0

    lax.fori_loop(0, nchunk, tok_row, 0)

    gbase = base

    def hash_step(k, _):
        cur = tok_v[pl.ds(8 + k * 16, 16)]
        prev = tok_v[pl.ds(7 + k * 16, 16)]
        h = (_MULT_A * cur ^ _MULT_B * prev) % _MOD
        pos = gbase + k * 16 + lax.iota(jnp.int32, 16)
        idx_v[pl.ds(k * 16, 16)] = jnp.where(pos % _SEQ == 0, _MOD, h)
        return 0

    lax.fori_loop(0, nchunk * _CHUNK // 16, hash_step, 0)

    def gather_step(c, _):
        pltpu.async_copy(
            table_hbm.at[idx_v.at[pl.ds(c * _CHUNK, _CHUNK)]], rows_v,
            sem).wait()
        pltpu.sync_copy(
            rows_v,
            out_hbm.at[pl.ds(base + c * _CHUNK, _CHUNK), pl.ds(0, _EDIM)])
        return 0

    lax.fori_loop(0, nchunk, gather_step, 0)


def _make_sc_gather(nrows):
    nchunk = nrows // (_NW * _CHUNK)
    mesh = plsc.VectorSubcoreMesh(
        core_axis_name="c", subcore_axis_name="s", num_cores=_NC,
        num_subcores=_NS)
    return pl.kernel(
        functools.partial(_sc_body, nchunk),
        out_type=jax.ShapeDtypeStruct((nrows, _PDIM), jnp.float32),
        mesh=mesh,
        scratch_types=[
            pltpu.VMEM((nchunk * _CHUNK + 8,), jnp.int32),
            pltpu.VMEM((nchunk * _CHUNK,), jnp.int32),
            pltpu.VMEM((_CHUNK, _EDIM), jnp.float32),
            pltpu.SemaphoreType.DMA,
        ],
        compiler_params=pltpu.CompilerParams(use_tc_tiling_on_sc=False),
    )


_RB = 1024


def _mm_body(h_ref, w_ref, o_ref):
    o_ref[...] = jnp.dot(h_ref[:, :_EDIM], w_ref[...],
                         preferred_element_type=jnp.float32)


def _tc_project(h, w):
    n = h.shape[0]
    return pl.pallas_call(
        _mm_body,
        grid=(n // _RB,),
        in_specs=[
            pl.BlockSpec((_RB, _PDIM), lambda i: (i, 0)),
            pl.BlockSpec((_EDIM, _MDIM), lambda i: (0, 0)),
        ],
        out_specs=pl.BlockSpec((_RB, _MDIM), lambda i: (i, 0)),
        out_shape=jax.ShapeDtypeStruct((n, _MDIM), jnp.float32),
    )(h, w)


_K = 4


@jax.jit
def _pipeline(token_ids, embed_weight, proj_weight, scale):
    ii = jnp.arange(_N, dtype=jnp.int32).reshape(_N // _PDIM, _PDIM)
    tok128 = token_ids[ii // _SEQ, ii % _SEQ]
    w = (proj_weight * scale).T
    nrows = _N // _K
    sc = _make_sc_gather(nrows)
    trows = nrows // _PDIM
    outs = []
    for k in range(_K):
        g = sc(lax.slice_in_dim(tok128, k * trows, (k + 1) * trows),
               embed_weight)
        outs.append(_tc_project(g, w))
    return jnp.concatenate(outs, axis=0).reshape(_BATCH, _SEQ, _MDIM)


def kernel(token_ids, embed_weight, proj_weight, scale):
    return _pipeline(token_ids, embed_weight, proj_weight, scale)


# speedup vs baseline: 1.1586x; 1.1586x over previous
import functools

import jax
import jax.numpy as jnp
from jax import lax
from jax.experimental import pallas as pl
from jax.experimental.pallas import tpu as pltpu
from jax.experimental.pallas import tpu_sc as plsc

_BATCH = 1024
_SEQ = 200
_N = _BATCH * _SEQ
_NC = 2
_NS = 16
_NW = _NC * _NS
_PER_W = _N // _NW
_CHUNK = 128
_NCHUNK = _PER_W // _CHUNK
_HVEC = _PER_W // 16
_EDIM = 64
_PDIM = 128
_MDIM = 512
_MULT_A = 36313
_MULT_B = 27191
_MOD = 999999


def _sc_body(nchunk, tok_hbm, table_hbm, out_hbm, tok_v, idx_v, rows_v, sem):
    wid = lax.axis_index("s") * _NC + lax.axis_index("c")
    base = wid * nchunk * _CHUNK

    def tok_row(r, _):
        pltpu.sync_copy(tok_hbm.at[wid * nchunk + r],
                        tok_v.at[pl.ds(8 + r * _PDIM, _PDIM)])
        return 0

    lax.fori_loop(0, nchunk, tok_row, 0)

    def hash_step(k, _):
        cur = tok_v[pl.ds(8 + k * 16, 16)]
        prev = tok_v[pl.ds(7 + k * 16, 16)]
        h = (_MULT_A * cur ^ _MULT_B * prev) % _MOD
        pos = k * 16 + lax.iota(jnp.int32, 16)
        idx_v[pl.ds(k * 16, 16)] = jnp.where(pos % _SEQ == 0, _MOD, h)
        return 0

    lax.fori_loop(0, nchunk * _CHUNK // 16, hash_step, 0)

    def gather_step(c, _):
        pltpu.async_copy(
            table_hbm.at[idx_v.at[pl.ds(c * _CHUNK, _CHUNK)]], rows_v,
            sem).wait()
        pltpu.sync_copy(
            rows_v,
            out_hbm.at[pl.ds(base + c * _CHUNK, _CHUNK), pl.ds(0, _EDIM)])
        return 0

    lax.fori_loop(0, nchunk, gather_step, 0)


def _make_sc_gather(nrows):
    nchunk = nrows // (_NW * _CHUNK)
    mesh = plsc.VectorSubcoreMesh(
        core_axis_name="c", subcore_axis_name="s", num_cores=_NC,
        num_subcores=_NS)
    return pl.kernel(
        functools.partial(_sc_body, nchunk),
        out_type=jax.ShapeDtypeStruct((nrows, _PDIM), jnp.float32),
        mesh=mesh,
        scratch_types=[
            pltpu.VMEM((nchunk * _CHUNK + 8,), jnp.int32),
            pltpu.VMEM((nchunk * _CHUNK,), jnp.int32),
            pltpu.VMEM((_CHUNK, _EDIM), jnp.float32),
            pltpu.SemaphoreType.DMA,
        ],
        compiler_params=pltpu.CompilerParams(use_tc_tiling_on_sc=False),
    )


_RB = 1024


def _mm_body(h_ref, w_ref, acc_ref, o_ref):
    del acc_ref
    o_ref[...] = jnp.dot(h_ref[:, :_EDIM], w_ref[...],
                         preferred_element_type=jnp.float32)


def _tc_project_part(h, w, acc, part):
    n = h.shape[0]
    off = part * (n // _RB)
    return pl.pallas_call(
        _mm_body,
        grid=(n // _RB,),
        in_specs=[
            pl.BlockSpec((_RB, _PDIM), lambda i: (i, 0)),
            pl.BlockSpec((_EDIM, _MDIM), lambda i: (0, 0)),
            pl.BlockSpec(memory_space=pl.MemorySpace.ANY),
        ],
        out_specs=pl.BlockSpec((_RB, _MDIM), lambda i: (off + i, 0)),
        out_shape=jax.ShapeDtypeStruct((_N, _MDIM), jnp.float32),
        input_output_aliases={2: 0},
    )(h, w, acc)


_K = 2


@jax.jit
def _pipeline(token_ids, embed_weight, proj_weight, scale):
    ii = jnp.arange(_N, dtype=jnp.int32).reshape(_N // _PDIM, _PDIM)
    tok128 = token_ids[ii // _SEQ, ii % _SEQ]
    w = (proj_weight * scale).T
    nrows = _N // _K
    sc = _make_sc_gather(nrows)
    trows = nrows // _PDIM
    gs = [sc(lax.slice_in_dim(tok128, k * trows, (k + 1) * trows),
             embed_weight) for k in range(_K)]
    acc = jnp.empty((_N, _MDIM), jnp.float32)
    for k in range(_K):
        acc = _tc_project_part(gs[k], w, acc, k)
    return acc.reshape(_BATCH, _SEQ, _MDIM)


def kernel(token_ids, embed_weight, proj_weight, scale):
    return _pipeline(token_ids, embed_weight, proj_weight, scale)


# speedup vs baseline: 1.3279x; 1.1461x over previous
import functools

import jax
import jax.numpy as jnp
from jax import lax
from jax.experimental import pallas as pl
from jax.experimental.pallas import tpu as pltpu
from jax.experimental.pallas import tpu_sc as plsc

_BATCH = 1024
_SEQ = 200
_N = _BATCH * _SEQ
_NC = 2
_NS = 16
_NW = _NC * _NS
_PER_W = _N // _NW
_CHUNK = 128
_NCHUNK = _PER_W // _CHUNK
_HVEC = _PER_W // 16
_EDIM = 64
_PDIM = 128
_MDIM = 512
_MULT_A = 36313
_MULT_B = 27191
_MOD = 999999


def _sc_body(nchunk, tok_hbm, table_hbm, out_hbm, tok_v, idx_v, rows_v, sem):
    wid = lax.axis_index("s") * _NC + lax.axis_index("c")
    base = wid * nchunk * _CHUNK

    def tok_row(r, _):
        pltpu.sync_copy(tok_hbm.at[wid * nchunk + r],
                        tok_v.at[pl.ds(8 + r * _PDIM, _PDIM)])
        return 0

    lax.fori_loop(0, nchunk, tok_row, 0)

    def hash_step(k, _):
        cur = tok_v[pl.ds(8 + k * 16, 16)]
        prev = tok_v[pl.ds(7 + k * 16, 16)]
        h = (_MULT_A * cur ^ _MULT_B * prev) % _MOD
        pos = k * 16 + lax.iota(jnp.int32, 16)
        idx_v[pl.ds(k * 16, 16)] = jnp.where(pos % _SEQ == 0, _MOD, h)
        return 0

    lax.fori_loop(0, nchunk * _CHUNK // 16, hash_step, 0)

    def gather_step(c, _):
        pltpu.async_copy(
            table_hbm.at[idx_v.at[pl.ds(c * _CHUNK, _CHUNK)]], rows_v,
            sem).wait()
        pltpu.sync_copy(
            rows_v,
            out_hbm.at[pl.ds(base + c * _CHUNK, _CHUNK), pl.ds(0, _EDIM)])
        return 0

    lax.fori_loop(0, nchunk, gather_step, 0)


def _make_sc_gather(nrows):
    nchunk = nrows // (_NW * _CHUNK)
    mesh = plsc.VectorSubcoreMesh(
        core_axis_name="c", subcore_axis_name="s", num_cores=_NC,
        num_subcores=_NS)
    return pl.kernel(
        functools.partial(_sc_body, nchunk),
        out_type=jax.ShapeDtypeStruct((nrows, _PDIM), jnp.float32),
        mesh=mesh,
        scratch_types=[
            pltpu.VMEM((nchunk * _CHUNK + 8,), jnp.int32),
            pltpu.VMEM((nchunk * _CHUNK,), jnp.int32),
            pltpu.VMEM((_CHUNK, _EDIM), jnp.float32),
            pltpu.SemaphoreType.DMA,
        ],
        compiler_params=pltpu.CompilerParams(use_tc_tiling_on_sc=False),
    )


_RB = 1024


def _mm_body(h_ref, w_ref, o_ref):
    o_ref[...] = jnp.dot(h_ref[:, :_EDIM], w_ref[...],
                         preferred_element_type=jnp.float32)


def _mm_body_acc(h_ref, w_ref, acc_ref, o_ref):
    del acc_ref
    o_ref[...] = jnp.dot(h_ref[:, :_EDIM], w_ref[...],
                         preferred_element_type=jnp.float32)


def _tc_project_part(h, w, acc, part):
    n = h.shape[0]
    off = part * (n // _RB)
    specs = [
        pl.BlockSpec((_RB, _PDIM), lambda i: (i, 0)),
        pl.BlockSpec((_EDIM, _MDIM), lambda i: (0, 0)),
    ]
    args = [h, w]
    body = _mm_body
    aliases = {}
    if part:
        specs.append(pl.BlockSpec(memory_space=pl.MemorySpace.ANY))
        args.append(acc)
        body = _mm_body_acc
        aliases = {2: 0}
    return pl.pallas_call(
        body,
        grid=(n // _RB,),
        in_specs=specs,
        out_specs=pl.BlockSpec((_RB, _MDIM), lambda i: (off + i, 0)),
        out_shape=jax.ShapeDtypeStruct((_N, _MDIM), jnp.float32),
        input_output_aliases=aliases,
    )(*args)


_K = 2


@jax.jit
def _pipeline(token_ids, embed_weight, proj_weight, scale):
    ii = jnp.arange(_N, dtype=jnp.int32).reshape(_N // _PDIM, _PDIM)
    tok128 = token_ids[ii // _SEQ, ii % _SEQ]
    w = (proj_weight * scale).T
    nrows = _N // _K
    sc = _make_sc_gather(nrows)
    trows = nrows // _PDIM
    gs = [sc(lax.slice_in_dim(tok128, k * trows, (k + 1) * trows),
             embed_weight) for k in range(_K)]
    acc = None
    for k in range(_K):
        acc = _tc_project_part(gs[k], w, acc, k)
    return acc.reshape(_BATCH, _SEQ, _MDIM)


def kernel(token_ids, embed_weight, proj_weight, scale):
    return _pipeline(token_ids, embed_weight, proj_weight, scale)


# speedup vs baseline: 1.3976x; 1.0525x over previous
import functools

import jax
import jax.numpy as jnp
from jax import lax
from jax.experimental import pallas as pl
from jax.experimental.pallas import tpu as pltpu
from jax.experimental.pallas import tpu_sc as plsc

_BATCH = 1024
_SEQ = 200
_N = _BATCH * _SEQ
_NC = 2
_NS = 16
_NW = _NC * _NS
_PER_W = _N // _NW
_CHUNK = 128
_NCHUNK = _PER_W // _CHUNK
_HVEC = _PER_W // 16
_EDIM = 64
_PDIM = 128
_MDIM = 512
_MULT_A = 36313
_MULT_B = 27191
_MOD = 999999


def _sc_body(nchunk, tok_hbm, table_hbm, out_hbm, tok_v, idx_v, rows_v, sem):
    wid = lax.axis_index("s") * _NC + lax.axis_index("c")
    base = wid * nchunk * _CHUNK

    def tok_row(r, _):
        pltpu.sync_copy(tok_hbm.at[wid * nchunk + r],
                        tok_v.at[pl.ds(8 + r * _PDIM, _PDIM)])
        return 0

    lax.fori_loop(0, nchunk, tok_row, 0)

    def hash_step(k, _):
        cur = tok_v[pl.ds(8 + k * 16, 16)]
        prev = tok_v[pl.ds(7 + k * 16, 16)]
        h = (_MULT_A * cur ^ _MULT_B * prev) % _MOD
        pos = k * 16 + lax.iota(jnp.int32, 16)
        idx_v[pl.ds(k * 16, 16)] = jnp.where(pos % _SEQ == 0, _MOD, h)
        return 0

    lax.fori_loop(0, nchunk * _CHUNK // 16, hash_step, 0)

    def gather_step(c, _):
        pltpu.async_copy(
            table_hbm.at[idx_v.at[pl.ds(c * _CHUNK, _CHUNK)]], rows_v,
            sem).wait()
        pltpu.sync_copy(
            rows_v,
            out_hbm.at[pl.ds(base + c * _CHUNK, _CHUNK), pl.ds(0, _EDIM)])
        return 0

    lax.fori_loop(0, nchunk, gather_step, 0)


def _make_sc_gather(nrows):
    nchunk = nrows // (_NW * _CHUNK)
    mesh = plsc.VectorSubcoreMesh(
        core_axis_name="c", subcore_axis_name="s", num_cores=_NC,
        num_subcores=_NS)
    return pl.kernel(
        functools.partial(_sc_body, nchunk),
        out_type=jax.ShapeDtypeStruct((nrows, _PDIM), jnp.float32),
        mesh=mesh,
        scratch_types=[
            pltpu.VMEM((nchunk * _CHUNK + 8,), jnp.int32),
            pltpu.VMEM((nchunk * _CHUNK,), jnp.int32),
            pltpu.VMEM((_CHUNK, _EDIM), jnp.float32),
            pltpu.SemaphoreType.DMA,
        ],
        compiler_params=pltpu.CompilerParams(use_tc_tiling_on_sc=False),
    )


_RB = 2048


def _mm_body(h_ref, w_ref, o_ref):
    o_ref[...] = jnp.dot(h_ref[:, :_EDIM], w_ref[...],
                         preferred_element_type=jnp.float32)


def _mm_body_acc(h_ref, w_ref, acc_ref, o_ref):
    del acc_ref
    o_ref[...] = jnp.dot(h_ref[:, :_EDIM], w_ref[...],
                         preferred_element_type=jnp.float32)


def _tc_project_part(h, w, acc, part):
    n = h.shape[0]
    off = part * (n // _RB)
    specs = [
        pl.BlockSpec((_RB, _PDIM), lambda i: (i, 0)),
        pl.BlockSpec((_EDIM, _MDIM), lambda i: (0, 0)),
    ]
    args = [h, w]
    body = _mm_body
    aliases = {}
    if part:
        specs.append(pl.BlockSpec(memory_space=pl.MemorySpace.ANY))
        args.append(acc)
        body = _mm_body_acc
        aliases = {2: 0}
    return pl.pallas_call(
        body,
        grid=(n // _RB,),
        in_specs=specs,
        out_specs=pl.BlockSpec((_RB, _MDIM), lambda i: (off + i, 0)),
        out_shape=jax.ShapeDtypeStruct((_N, _MDIM), jnp.float32),
        input_output_aliases=aliases,
    )(*args)


_K = 2


@jax.jit
def _pipeline(token_ids, embed_weight, proj_weight, scale):
    ii = jnp.arange(_N, dtype=jnp.int32).reshape(_N // _PDIM, _PDIM)
    tok128 = token_ids[ii // _SEQ, ii % _SEQ]
    w = (proj_weight * scale).T
    nrows = _N // _K
    sc = _make_sc_gather(nrows)
    trows = nrows // _PDIM
    gs = [sc(lax.slice_in_dim(tok128, k * trows, (k + 1) * trows),
             embed_weight) for k in range(_K)]
    acc = None
    for k in range(_K):
        acc = _tc_project_part(gs[k], w, acc, k)
    return acc.reshape(_BATCH, _SEQ, _MDIM)


def kernel(token_ids, embed_weight, proj_weight, scale):
    return _pipeline(token_ids, embed_weight, proj_weight, scale)


# speedup vs baseline: 1.4080x; 1.0075x over previous
import functools

import jax
import jax.numpy as jnp
from jax import lax
from jax.experimental import pallas as pl
from jax.experimental.pallas import tpu as pltpu
from jax.experimental.pallas import tpu_sc as plsc

_BATCH = 1024
_SEQ = 200
_N = _BATCH * _SEQ
_NC = 2
_NS = 16
_NW = _NC * _NS
_PER_W = _N // _NW
_CHUNK = 128
_NCHUNK = _PER_W // _CHUNK
_HVEC = _PER_W // 16
_EDIM = 64
_PDIM = 128
_MDIM = 512
_MULT_A = 36313
_MULT_B = 27191
_MOD = 999999


def _sc_body(nchunk, tok_hbm, table_hbm, out_hbm, tok_v, idx_v, rows_v, sem):
    wid = lax.axis_index("s") * _NC + lax.axis_index("c")
    base = wid * nchunk * _CHUNK

    def tok_row(r, _):
        pltpu.sync_copy(tok_hbm.at[wid * nchunk + r],
                        tok_v.at[pl.ds(8 + r * _PDIM, _PDIM)])
        return 0

    lax.fori_loop(0, nchunk, tok_row, 0)

    def hash_step(k, _):
        cur = tok_v[pl.ds(8 + k * 16, 16)]
        prev = tok_v[pl.ds(7 + k * 16, 16)]
        h = (_MULT_A * cur ^ _MULT_B * prev) % _MOD
        pos = k * 16 + lax.iota(jnp.int32, 16)
        idx_v[pl.ds(k * 16, 16)] = jnp.where(pos % _SEQ == 0, _MOD, h)
        return 0

    lax.fori_loop(0, nchunk * _CHUNK // 16, hash_step, 0)

    def gather_step(c, _):
        pltpu.async_copy(
            table_hbm.at[idx_v.at[pl.ds(c * _CHUNK, _CHUNK)]], rows_v,
            sem).wait()
        pltpu.sync_copy(
            rows_v,
            out_hbm.at[pl.ds(base + c * _CHUNK, _CHUNK), pl.ds(0, _EDIM)])
        return 0

    lax.fori_loop(0, nchunk, gather_step, 0)


def _make_sc_gather(nrows):
    nchunk = nrows // (_NW * _CHUNK)
    mesh = plsc.VectorSubcoreMesh(
        core_axis_name="c", subcore_axis_name="s", num_cores=_NC,
        num_subcores=_NS)
    return pl.kernel(
        functools.partial(_sc_body, nchunk),
        out_type=jax.ShapeDtypeStruct((nrows, _PDIM), jnp.float32),
        mesh=mesh,
        scratch_types=[
            pltpu.VMEM((nchunk * _CHUNK + 8,), jnp.int32),
            pltpu.VMEM((nchunk * _CHUNK,), jnp.int32),
            pltpu.VMEM((_CHUNK, _EDIM), jnp.float32),
            pltpu.SemaphoreType.DMA,
        ],
        compiler_params=pltpu.CompilerParams(use_tc_tiling_on_sc=False),
    )


_RB = 4096


def _mm_body(h_ref, w_ref, o_ref):
    o_ref[...] = jnp.dot(h_ref[:, :_EDIM], w_ref[...],
                         preferred_element_type=jnp.float32)


def _mm_body_acc(h_ref, w_ref, acc_ref, o_ref):
    del acc_ref
    o_ref[...] = jnp.dot(h_ref[:, :_EDIM], w_ref[...],
                         preferred_element_type=jnp.float32)


def _tc_project_part(h, w, acc, part):
    n = h.shape[0]
    off = part * (n // _RB)
    specs = [
        pl.BlockSpec((_RB, _PDIM), lambda i: (i, 0)),
        pl.BlockSpec((_EDIM, _MDIM), lambda i: (0, 0)),
    ]
    args = [h, w]
    body = _mm_body
    aliases = {}
    if part:
        specs.append(pl.BlockSpec(memory_space=pl.MemorySpace.ANY))
        args.append(acc)
        body = _mm_body_acc
        aliases = {2: 0}
    return pl.pallas_call(
        body,
        grid=(n // _RB,),
        in_specs=specs,
        out_specs=pl.BlockSpec((_RB, _MDIM), lambda i: (off + i, 0)),
        out_shape=jax.ShapeDtypeStruct((_N, _MDIM), jnp.float32),
        input_output_aliases=aliases,
    )(*args)


_K = 2


@jax.jit
def _pipeline(token_ids, embed_weight, proj_weight, scale):
    ii = jnp.arange(_N, dtype=jnp.int32).reshape(_N // _PDIM, _PDIM)
    tok128 = token_ids[ii // _SEQ, ii % _SEQ]
    w = (proj_weight * scale).T
    nrows = _N // _K
    sc = _make_sc_gather(nrows)
    trows = nrows // _PDIM
    gs = [sc(lax.slice_in_dim(tok128, k * trows, (k + 1) * trows),
             embed_weight) for k in range(_K)]
    acc = None
    for k in range(_K):
        acc = _tc_project_part(gs[k], w, acc, k)
    return acc.reshape(_BATCH, _SEQ, _MDIM)


def kernel(token_ids, embed_weight, proj_weight, scale):
    return _pipeline(token_ids, embed_weight, proj_weight, scale)


# speedup vs baseline: 1.4155x; 1.0053x over previous
import functools

import jax
import jax.numpy as jnp
from jax import lax
from jax.experimental import pallas as pl
from jax.experimental.pallas import tpu as pltpu
from jax.experimental.pallas import tpu_sc as plsc

_BATCH = 1024
_SEQ = 200
_N = _BATCH * _SEQ
_NC = 2
_NS = 16
_NW = _NC * _NS
_PER_W = _N // _NW
_CHUNK = 128
_NCHUNK = _PER_W // _CHUNK
_HVEC = _PER_W // 16
_EDIM = 64
_PDIM = 128
_MDIM = 512
_MULT_A = 36313
_MULT_B = 27191
_MOD = 999999


def _sc_body(nchunk, tok_hbm, table_hbm, out_hbm, tok_v, idx_v, rows_v, sem):
    wid = lax.axis_index("s") * _NC + lax.axis_index("c")
    base = wid * nchunk * _CHUNK

    def tok_row(r, _):
        pltpu.sync_copy(tok_hbm.at[wid * nchunk + r],
                        tok_v.at[pl.ds(8 + r * _PDIM, _PDIM)])
        return 0

    lax.fori_loop(0, nchunk, tok_row, 0)

    def hash_step(k, _):
        cur = tok_v[pl.ds(8 + k * 16, 16)]
        prev = tok_v[pl.ds(7 + k * 16, 16)]
        h = (_MULT_A * cur ^ _MULT_B * prev) % _MOD
        pos = k * 16 + lax.iota(jnp.int32, 16)
        idx_v[pl.ds(k * 16, 16)] = jnp.where(pos % _SEQ == 0, _MOD, h)
        return 0

    lax.fori_loop(0, nchunk * _CHUNK // 16, hash_step, 0)

    def gather_step(c, _):
        pltpu.async_copy(
            table_hbm.at[idx_v.at[pl.ds(c * _CHUNK, _CHUNK)]], rows_v,
            sem).wait()
        pltpu.sync_copy(
            rows_v,
            out_hbm.at[pl.ds(base + c * _CHUNK, _CHUNK), pl.ds(0, _EDIM)])
        return 0

    lax.fori_loop(0, nchunk, gather_step, 0)


def _make_sc_gather(nrows):
    nchunk = nrows // (_NW * _CHUNK)
    mesh = plsc.VectorSubcoreMesh(
        core_axis_name="c", subcore_axis_name="s", num_cores=_NC,
        num_subcores=_NS)
    return pl.kernel(
        functools.partial(_sc_body, nchunk),
        out_type=jax.ShapeDtypeStruct((nrows, _PDIM), jnp.float32),
        mesh=mesh,
        scratch_types=[
            pltpu.VMEM((nchunk * _CHUNK + 8,), jnp.int32),
            pltpu.VMEM((nchunk * _CHUNK,), jnp.int32),
            pltpu.VMEM((_CHUNK, _EDIM), jnp.float32),
            pltpu.SemaphoreType.DMA,
        ],
        compiler_params=pltpu.CompilerParams(use_tc_tiling_on_sc=False),
    )


_RB = 8192


def _mm_body(h_ref, w_ref, o_ref):
    o_ref[...] = jnp.dot(h_ref[:, :_EDIM], w_ref[...],
                         preferred_element_type=jnp.float32)


def _mm_body_acc(h_ref, w_ref, acc_ref, o_ref):
    del acc_ref
    o_ref[...] = jnp.dot(h_ref[:, :_EDIM], w_ref[...],
                         preferred_element_type=jnp.float32)


def _tc_project_part(h, w, acc, part):
    n = h.shape[0]
    off = part * (n // _RB)
    specs = [
        pl.BlockSpec((_RB, _PDIM), lambda i: (i, 0)),
        pl.BlockSpec((_EDIM, _MDIM), lambda i: (0, 0)),
    ]
    args = [h, w]
    body = _mm_body
    aliases = {}
    if part:
        specs.append(pl.BlockSpec(memory_space=pl.MemorySpace.ANY))
        args.append(acc)
        body = _mm_body_acc
        aliases = {2: 0}
    return pl.pallas_call(
        body,
        grid=(n // _RB,),
        in_specs=specs,
        out_specs=pl.BlockSpec((_RB, _MDIM), lambda i: (off + i, 0)),
        out_shape=jax.ShapeDtypeStruct((_N, _MDIM), jnp.float32),
        input_output_aliases=aliases,
    )(*args)


_K = 2


@jax.jit
def _pipeline(token_ids, embed_weight, proj_weight, scale):
    ii = jnp.arange(_N, dtype=jnp.int32).reshape(_N // _PDIM, _PDIM)
    tok128 = token_ids[ii // _SEQ, ii % _SEQ]
    w = (proj_weight * scale).T
    nrows = _N // _K
    sc = _make_sc_gather(nrows)
    trows = nrows // _PDIM
    gs = [sc(lax.slice_in_dim(tok128, k * trows, (k + 1) * trows),
             embed_weight) for k in range(_K)]
    acc = None
    for k in range(_K):
        acc = _tc_project_part(gs[k], w, acc, k)
    return acc.reshape(_BATCH, _SEQ, _MDIM)


def kernel(token_ids, embed_weight, proj_weight, scale):
    return _pipeline(token_ids, embed_weight, proj_weight, scale)
